# baseline (speedup 1.0000x reference)
import functools

import jax
import jax.numpy as jnp
from jax import lax
from jax.experimental import pallas as pl
from jax.experimental.pallas import tpu as pltpu
from jax.experimental.pallas import tpu_sc as plsc


_C1 = 8192


def _transform_body(t_ref, w_ref, b_ref, o_ref):
    y = lax.dot_general(
        t_ref[...], w_ref[...],
        (((0,), (1,)), ((), ())),
        preferred_element_type=jnp.float32,
    ) + b_ref[...]
    o_ref[:, 0:64] = y
    o_ref[:, 64:128] = y


def _transform_table(table_t, W, b2):
    H, V = table_t.shape
    grid = (V + _C1 - 1) // _C1
    return pl.pallas_call(
        _transform_body,
        grid=(grid,),
        in_specs=[
            pl.BlockSpec((H, _C1), lambda i: (0, i)),
            pl.BlockSpec((H, H), lambda i: (0, 0)),
            pl.BlockSpec((1, H), lambda i: (0, 0)),
        ],
        out_specs=pl.BlockSpec((_C1, 2 * H), lambda i: (i, 0)),
        out_shape=jax.ShapeDtypeStruct((V, 2 * H), jnp.float32),
    )(table_t, W, b2)


_NC = 2
_NS = 16
_NW = _NC * _NS


def _make_gather(L, B, H):
    N = B * L
    bw = B // _NW
    npair = L // 2
    cw = 2 * bw

    mesh = plsc.VectorSubcoreMesh(core_axis_name="c", subcore_axis_name="s")

    @functools.partial(
        pl.kernel,
        mesh=mesh,
        out_type=jax.ShapeDtypeStruct((N, H), jnp.float32),
        compiler_params=pltpu.CompilerParams(use_tc_tiling_on_sc=False),
        scratch_types=[
            pltpu.VMEM((L, bw), jnp.int32),
            pltpu.VMEM((cw, H), jnp.float32),
            pltpu.SemaphoreType.DMA,
        ],
    )
    def gather_k(t2_hbm, seq2_hbm, out_hbm, idx2d, rows_v, sem):
        wid = lax.axis_index("s") * _NC + lax.axis_index("c")
        b0 = wid * bw
        pltpu.sync_copy(seq2_hbm.at[:, pl.ds(b0, bw)], idx2d)

        def body(lp, carry):
            cp1 = pltpu.async_copy(
                t2_hbm.at[idx2d.at[2 * lp]],
                rows_v.at[pl.ds(0, bw)], sem)
            cp2 = pltpu.async_copy(
                t2_hbm.at[idx2d.at[2 * lp + 1]],
                rows_v.at[pl.ds(bw, bw)], sem)
            cp1.wait()
            cp2.wait()
            pltpu.sync_copy(rows_v, out_hbm.at[pl.ds(lp * 2 * B + wid * cw, cw)])
            return carry

        lax.fori_loop(0, npair, body, 0)

    return gather_k


def _xpose_body(g_ref, o_ref):
    x = g_ref[0, :, 0]
    o_ref[0] = x.reshape(-1, x.shape[2]).T


def _transpose_out(g5, L, B, H):
    nw, bw = g5.shape[1], g5.shape[3]
    return pl.pallas_call(
        _xpose_body,
        grid=(L // 2, 2),
        in_specs=[pl.BlockSpec(
            (1, nw, 1, bw, H), lambda i, e: (i, 0, e, 0, 0))],
        out_specs=pl.BlockSpec((1, H, B), lambda i, e: (2 * i + e, 0, 0)),
        out_shape=jax.ShapeDtypeStruct((L, H, B), jnp.float32),
    )(g5)


def kernel(seq, table, W, b):
    B, L = seq.shape
    V, H = table.shape
    t2d = _transform_table(table.T, W, b.reshape(1, H))
    t2v = t2d.reshape(2 * V, H)
    seq2_t = seq.T * 2
    g = _make_gather(L, B, H)(t2v, seq2_t)
    g5 = g.reshape(L // 2, _NW, 2, B // _NW, H)
    out_t = _transpose_out(g5, L, B, H)
    return out_t.transpose(2, 0, 1)


# speedup vs baseline: 1.5270x; 1.5270x over previous
import functools

import jax
import jax.numpy as jnp
from jax import lax
from jax.experimental import pallas as pl
from jax.experimental.pallas import tpu as pltpu
from jax.experimental.pallas import tpu_sc as plsc


_C1 = 8192


def _transform_body(t_ref, w_ref, b_ref, o_ref):
    y = lax.dot_general(
        t_ref[...], w_ref[...],
        (((0,), (1,)), ((), ())),
        preferred_element_type=jnp.float32,
    ) + b_ref[...]
    o_ref[:, 0:64] = y
    o_ref[:, 64:128] = y


def _transform_table(table_t, W, b2):
    H, V = table_t.shape
    grid = (V + _C1 - 1) // _C1
    return pl.pallas_call(
        _transform_body,
        grid=(grid,),
        in_specs=[
            pl.BlockSpec((H, _C1), lambda i: (0, i)),
            pl.BlockSpec((H, H), lambda i: (0, 0)),
            pl.BlockSpec((1, H), lambda i: (0, 0)),
        ],
        out_specs=pl.BlockSpec((_C1, 2 * H), lambda i: (i, 0)),
        out_shape=jax.ShapeDtypeStruct((V, 2 * H), jnp.float32),
    )(table_t, W, b2)


_NC = 2
_NS = 16
_NW = _NC * _NS


def _make_gather(L, B, H):
    N = B * L
    bw = B // _NW
    npair = L // 2
    cw = 2 * bw

    mesh = plsc.VectorSubcoreMesh(core_axis_name="c", subcore_axis_name="s")

    @functools.partial(
        pl.kernel,
        mesh=mesh,
        out_type=jax.ShapeDtypeStruct((N // 2, 2, H), jnp.float32),
        compiler_params=pltpu.CompilerParams(use_tc_tiling_on_sc=False),
        scratch_types=[
            pltpu.VMEM((L, bw), jnp.int32),
            pltpu.VMEM((cw, H), jnp.float32),
            pltpu.SemaphoreType.DMA,
        ],
    )
    def gather_k(t2_hbm, seq2_hbm, out_hbm, idx2d, rows_v, sem):
        wid = lax.axis_index("s") * _NC + lax.axis_index("c")
        b0 = wid * bw
        pltpu.sync_copy(seq2_hbm.at[:, pl.ds(b0, bw)], idx2d)

        def body(lp, carry):
            cp1 = pltpu.async_copy(
                t2_hbm.at[idx2d.at[2 * lp]], rows_v.at[pl.ds(0, bw)], sem)
            cp2 = pltpu.async_copy(
                t2_hbm.at[idx2d.at[2 * lp + 1]], rows_v.at[pl.ds(bw, bw)], sem)
            k0 = lp * B + wid * bw
            cp1.wait()
            pltpu.sync_copy(
                rows_v.at[pl.ds(0, bw)], out_hbm.at[pl.ds(k0, bw), 0])
            cp2.wait()
            pltpu.sync_copy(
                rows_v.at[pl.ds(bw, bw)], out_hbm.at[pl.ds(k0, bw), 1])
            return carry

        lax.fori_loop(0, npair, body, 0)

    return gather_k


def _xpose_body(g_ref, o_ref):
    x = g_ref[0]
    o_ref[0] = x[:, 0:64].T
    o_ref[1] = x[:, 64:128].T


def _transpose_out(g3, L, B, H):
    return pl.pallas_call(
        _xpose_body,
        grid=(L // 2,),
        in_specs=[pl.BlockSpec((1, B, 2 * H), lambda i: (i, 0, 0))],
        out_specs=pl.BlockSpec((2, H, B), lambda i: (i, 0, 0)),
        out_shape=jax.ShapeDtypeStruct((L, H, B), jnp.float32),
    )(g3)


def kernel(seq, table, W, b):
    B, L = seq.shape
    V, H = table.shape
    t2d = _transform_table(table.T, W, b.reshape(1, H))
    t2v = t2d.reshape(2 * V, H)
    seq2_t = seq.T * 2
    g = _make_gather(L, B, H)(t2v, seq2_t)
    g3 = g.reshape(L // 2, B, 2 * H)
    out_t = _transpose_out(g3, L, B, H)
    return out_t.transpose(2, 0, 1)


# speedup vs baseline: 1.7060x; 1.1172x over previous
import functools

import jax
import jax.numpy as jnp
from jax import lax
from jax.experimental import pallas as pl
from jax.experimental.pallas import tpu as pltpu
from jax.experimental.pallas import tpu_sc as plsc


_C1 = 8192


def _transform_body(t_ref, w_ref, b_ref, o_ref):
    y = lax.dot_general(
        t_ref[...], w_ref[...],
        (((0,), (1,)), ((), ())),
        preferred_element_type=jnp.float32,
    ) + b_ref[...]
    o_ref[:, 0:64] = y
    o_ref[:, 64:128] = y


def _transform_table(table_t, W, b2):
    H, V = table_t.shape
    grid = (V + _C1 - 1) // _C1
    return pl.pallas_call(
        _transform_body,
        grid=(grid,),
        in_specs=[
            pl.BlockSpec((H, _C1), lambda i: (0, i)),
            pl.BlockSpec((H, H), lambda i: (0, 0)),
            pl.BlockSpec((1, H), lambda i: (0, 0)),
        ],
        out_specs=pl.BlockSpec((_C1, 2 * H), lambda i: (i, 0)),
        out_shape=jax.ShapeDtypeStruct((V, 2 * H), jnp.float32),
    )(table_t, W, b2)


_NC = 2
_NS = 16
_NW = _NC * _NS


def _make_gather(L, B, H):
    N = B * L
    bw = B // _NW
    npair = L // 2
    cw = 2 * bw

    mesh = plsc.VectorSubcoreMesh(core_axis_name="c", subcore_axis_name="s")

    @functools.partial(
        pl.kernel,
        mesh=mesh,
        out_type=jax.ShapeDtypeStruct((N // 2, 2, H), jnp.float32),
        compiler_params=pltpu.CompilerParams(use_tc_tiling_on_sc=False),
        scratch_types=[
            pltpu.VMEM((L, bw), jnp.int32),
            pltpu.VMEM((cw, H), jnp.float32),
            pltpu.SemaphoreType.DMA,
        ],
    )
    def gather_k(t2_hbm, seq2_hbm, out_hbm, idx2d, rows_v, sem):
        wid = lax.axis_index("s") * _NC + lax.axis_index("c")
        b0 = wid * bw
        pltpu.sync_copy(seq2_hbm.at[:, pl.ds(b0, bw)], idx2d)

        def body(lp, carry):
            cp1 = pltpu.async_copy(
                t2_hbm.at[idx2d.at[2 * lp]], rows_v.at[pl.ds(0, bw)], sem)
            cp2 = pltpu.async_copy(
                t2_hbm.at[idx2d.at[2 * lp + 1]], rows_v.at[pl.ds(bw, bw)], sem)
            k0 = lp * B + wid * bw
            cp1.wait()
            pltpu.sync_copy(
                rows_v.at[pl.ds(0, bw)], out_hbm.at[pl.ds(k0, bw), 0])
            cp2.wait()
            pltpu.sync_copy(
                rows_v.at[pl.ds(bw, bw)], out_hbm.at[pl.ds(k0, bw), 1])
            return carry

        lax.fori_loop(0, npair, body, 0)

    return gather_k


def _xpose_body(g_ref, o_ref):
    x = g_ref[0]
    o_ref[0] = x[:, 0:64].T
    o_ref[1] = x[:, 64:128].T


def _xpose_body_acc(big_ref, g_ref, o_ref):
    del big_ref
    _xpose_body(g_ref, o_ref)


def _transpose_slab(g3, prev, s, Ls, L, B, H):
    base = s * Ls // 2
    out_spec = pl.BlockSpec((2, H, B), lambda i: (base + i, 0, 0))
    g_spec = pl.BlockSpec((1, B, 2 * H), lambda i: (i, 0, 0))
    out_shape = jax.ShapeDtypeStruct((L, H, B), jnp.float32)
    if prev is None:
        return pl.pallas_call(
            _xpose_body,
            grid=(Ls // 2,),
            in_specs=[g_spec],
            out_specs=out_spec,
            out_shape=out_shape,
        )(g3)
    return pl.pallas_call(
        _xpose_body_acc,
        grid=(Ls // 2,),
        in_specs=[pl.BlockSpec(memory_space=pl.ANY), g_spec],
        out_specs=out_spec,
        out_shape=out_shape,
        input_output_aliases={0: 0},
    )(prev, g3)


_S = 4


def kernel(seq, table, W, b):
    B, L = seq.shape
    V, H = table.shape
    t2d = _transform_table(table.T, W, b.reshape(1, H))
    t2v = t2d.reshape(2 * V, H)
    seq2_t = seq.T * 2
    Ls = L // _S
    gather = _make_gather(Ls, B, H)
    out_t = None
    for s in range(_S):
        g = gather(t2v, lax.slice_in_dim(seq2_t, s * Ls, (s + 1) * Ls))
        g3 = g.reshape(Ls // 2, B, 2 * H)
        out_t = _transpose_slab(g3, out_t, s, Ls, L, B, H)
    return out_t.transpose(2, 0, 1)


# speedup vs baseline: 1.7968x; 1.0533x over previous
import functools

import jax
import jax.numpy as jnp
from jax import lax
from jax.experimental import pallas as pl
from jax.experimental.pallas import tpu as pltpu
from jax.experimental.pallas import tpu_sc as plsc


_C1 = 16384


def _transform_body(t_ref, w_ref, b_ref, o_ref):
    y = lax.dot_general(
        t_ref[...], w_ref[...],
        (((0,), (1,)), ((), ())),
        preferred_element_type=jnp.float32,
    ) + b_ref[...]
    o_ref[:, 0:64] = y
    o_ref[:, 64:128] = y


def _transform_table(table_t, W, b2):
    H, V = table_t.shape
    grid = (V + _C1 - 1) // _C1
    return pl.pallas_call(
        _transform_body,
        grid=(grid,),
        in_specs=[
            pl.BlockSpec((H, _C1), lambda i: (0, i)),
            pl.BlockSpec((H, H), lambda i: (0, 0)),
            pl.BlockSpec((1, H), lambda i: (0, 0)),
        ],
        out_specs=pl.BlockSpec((_C1, 2 * H), lambda i: (i, 0)),
        out_shape=jax.ShapeDtypeStruct((V, 2 * H), jnp.float32),
    )(table_t, W, b2)


_NC = 2
_NS = 16
_NW = _NC * _NS


def _make_gather(L, B, H):
    N = B * L
    bw = B // _NW
    npair = L // 2
    cw = 2 * bw

    mesh = plsc.VectorSubcoreMesh(core_axis_name="c", subcore_axis_name="s")

    @functools.partial(
        pl.kernel,
        mesh=mesh,
        out_type=jax.ShapeDtypeStruct((N // 2, 2, H), jnp.float32),
        compiler_params=pltpu.CompilerParams(use_tc_tiling_on_sc=False),
        scratch_types=[
            pltpu.VMEM((L, bw), jnp.int32),
            pltpu.VMEM((cw, H), jnp.float32),
            pltpu.SemaphoreType.DMA,
        ],
    )
    def gather_k(t2_hbm, seq2_hbm, out_hbm, idx2d, rows_v, sem):
        wid = lax.axis_index("s") * _NC + lax.axis_index("c")
        b0 = wid * bw
        pltpu.sync_copy(seq2_hbm.at[:, pl.ds(b0, bw)], idx2d)

        def body(lp, carry):
            cp1 = pltpu.async_copy(
                t2_hbm.at[idx2d.at[2 * lp]], rows_v.at[pl.ds(0, bw)], sem)
            cp2 = pltpu.async_copy(
                t2_hbm.at[idx2d.at[2 * lp + 1]], rows_v.at[pl.ds(bw, bw)], sem)
            k0 = lp * B + wid * bw
            cp1.wait()
            pltpu.sync_copy(
                rows_v.at[pl.ds(0, bw)], out_hbm.at[pl.ds(k0, bw), 0])
            cp2.wait()
            pltpu.sync_copy(
                rows_v.at[pl.ds(bw, bw)], out_hbm.at[pl.ds(k0, bw), 1])
            return carry

        lax.fori_loop(0, npair, body, 0)

    return gather_k


def _xpose_body(g_ref, o_ref):
    x = g_ref[0]
    o_ref[0] = x[:, 0:64].T
    o_ref[1] = x[:, 64:128].T


def _xpose_body_acc(big_ref, g_ref, o_ref):
    del big_ref
    _xpose_body(g_ref, o_ref)


def _transpose_slab(g3, prev, s, Ls, L, B, H):
    base = s * Ls // 2
    out_spec = pl.BlockSpec((2, H, B), lambda i: (base + i, 0, 0))
    g_spec = pl.BlockSpec((1, B, 2 * H), lambda i: (i, 0, 0))
    out_shape = jax.ShapeDtypeStruct((L, H, B), jnp.float32)
    if prev is None:
        return pl.pallas_call(
            _xpose_body,
            grid=(Ls // 2,),
            in_specs=[g_spec],
            out_specs=out_spec,
            out_shape=out_shape,
        )(g3)
    return pl.pallas_call(
        _xpose_body_acc,
        grid=(Ls // 2,),
        in_specs=[pl.BlockSpec(memory_space=pl.ANY), g_spec],
        out_specs=out_spec,
        out_shape=out_shape,
        input_output_aliases={0: 0},
    )(prev, g3)


_S = 4


def kernel(seq, table, W, b):
    B, L = seq.shape
    V, H = table.shape
    t2d = _transform_table(table.T, W, b.reshape(1, H))
    t2v = t2d.reshape(2 * V, H)
    seq2_t = seq.T * 2
    Ls = L // _S
    gather = _make_gather(Ls, B, H)
    out_t = None
    for s in range(_S):
        g = gather(t2v, lax.slice_in_dim(seq2_t, s * Ls, (s + 1) * Ls))
        g3 = g.reshape(Ls // 2, B, 2 * H)
        out_t = _transpose_slab(g3, out_t, s, Ls, L, B, H)
    return out_t.transpose(2, 0, 1)
